# Initial kernel scaffold; baseline (speedup 1.0000x reference)
import jax
import jax.numpy as jnp
from jax.experimental import pallas as pl


def kernel(x_in, hyperedge_index, weight_node, bias, w_hat, s_e):
    raise NotImplementedError("write your pallas kernel here")


# speedup vs baseline: 465.2089x; 465.2089x over previous
import functools

import jax
import jax.numpy as jnp
from jax import lax
from jax.experimental import pallas as pl
from jax.experimental.pallas import tpu as pltpu
from jax.experimental.pallas import tpu_sc as plsc

N_NODES = 100000
N_EDGES = 100000
N_CONN = 6400000
W_MAX = 1.0

NC = 2
NS = 16
LANES = 16

NW = NC * NS
PER_W = N_CONN // NW
CHUNK = 20000
N_CHUNKS = PER_W // CHUNK
TPAD = 102400
TSLICE = TPAD // NS


def _fill(ref, value, n):
    def body(i, _):
        ref[pl.ds(i * LANES, LANES)] = jnp.full((LANES,), value, jnp.float32)
        return 0
    lax.fori_loop(0, n // LANES, body, 0)


_vmesh = plsc.VectorSubcoreMesh(core_axis_name="c", subcore_axis_name="s")


@functools.partial(
    pl.kernel,
    mesh=_vmesh,
    out_type=[
        jax.ShapeDtypeStruct((NC * TPAD,), jnp.float32),
        jax.ShapeDtypeStruct((NC * TPAD,), jnp.float32),
    ],
    scratch_types=[
        pltpu.VMEM((TSLICE,), jnp.float32),
        pltpu.VMEM((CHUNK,), jnp.int32),
        pltpu.VMEM((CHUNK,), jnp.int32),
        pltpu.VMEM((CHUNK,), jnp.float32),
        pltpu.VMEM((CHUNK,), jnp.float32),
        pltpu.VMEM_SHARED((TPAD,), jnp.float32),
        pltpu.VMEM_SHARED((TPAD,), jnp.float32),
        pltpu.VMEM_SHARED((TPAD,), jnp.float32),
        pltpu.SemaphoreType.DMA,
    ],
)
def _edge_accum(x_hbm, he_hbm, s_out, c_out, stage_v, nidx_v, eidx_v,
                vals_v, ones_v, x_sp, s_sp, c_sp, sem):
    cid = lax.axis_index("c")
    sid = lax.axis_index("s")
    wid = sid * NC + cid
    tlo = sid * TSLICE

    _fill(stage_v, 0.0, TSLICE)
    pltpu.sync_copy(stage_v, s_sp.at[pl.ds(tlo, TSLICE)])
    pltpu.sync_copy(stage_v, c_sp.at[pl.ds(tlo, TSLICE)])
    _fill(ones_v, 1.0, CHUNK)
    pltpu.sync_copy(x_hbm.at[pl.ds(tlo, TSLICE)], stage_v)
    pltpu.sync_copy(stage_v, x_sp.at[pl.ds(tlo, TSLICE)])
    plsc.subcore_barrier()

    def chunk_body(i, _):
        base = wid * PER_W + i * CHUNK
        pltpu.sync_copy(he_hbm.at[pl.ds(base, CHUNK)], nidx_v)
        pltpu.sync_copy(he_hbm.at[pl.ds(N_CONN + base, CHUNK)], eidx_v)
        pltpu.async_copy(x_sp.at[nidx_v], vals_v, sem).wait()
        pltpu.sync_copy(vals_v, s_sp.at[eidx_v], add=True)
        pltpu.sync_copy(ones_v, c_sp.at[eidx_v], add=True)
        return 0

    lax.fori_loop(0, N_CHUNKS, chunk_body, 0)
    plsc.subcore_barrier()

    pltpu.sync_copy(s_sp.at[pl.ds(tlo, TSLICE)], stage_v)
    pltpu.sync_copy(stage_v, s_out.at[pl.ds(cid * TPAD + tlo, TSLICE)])
    pltpu.sync_copy(c_sp.at[pl.ds(tlo, TSLICE)], stage_v)
    pltpu.sync_copy(stage_v, c_out.at[pl.ds(cid * TPAD + tlo, TSLICE)])


@functools.partial(
    pl.kernel,
    mesh=_vmesh,
    out_type=jax.ShapeDtypeStruct((NC * TPAD,), jnp.float32),
    scratch_types=[
        pltpu.VMEM((TSLICE,), jnp.float32),
        pltpu.VMEM((TSLICE,), jnp.float32),
        pltpu.VMEM((TSLICE,), jnp.float32),
        pltpu.VMEM((TSLICE,), jnp.float32),
        pltpu.VMEM((TSLICE,), jnp.float32),
        pltpu.VMEM((CHUNK,), jnp.int32),
        pltpu.VMEM((CHUNK,), jnp.int32),
        pltpu.VMEM((CHUNK,), jnp.float32),
        pltpu.VMEM_SHARED((TPAD,), jnp.float32),
        pltpu.VMEM_SHARED((TPAD,), jnp.float32),
        pltpu.SemaphoreType.DMA,
    ],
)
def _node_scatter(he_hbm, s_hbm, c_hbm, wh_hbm, se_hbm, out_p, buf_a, buf_b,
                  buf_c, buf_d, g_v, nidx_v, eidx_v, vals_v, g_sp, o_sp, sem):
    cid = lax.axis_index("c")
    sid = lax.axis_index("s")
    wid = sid * NC + cid
    tlo = sid * TSLICE

    pltpu.sync_copy(s_hbm.at[pl.ds(tlo, TSLICE)], buf_a)
    pltpu.sync_copy(s_hbm.at[pl.ds(TPAD + tlo, TSLICE)], buf_b)

    def sum_body(i, _):
        dsl = pl.ds(i * LANES, LANES)
        buf_a[dsl] = buf_a[dsl] + buf_b[dsl]
        return 0

    lax.fori_loop(0, TSLICE // LANES, sum_body, 0)
    pltpu.sync_copy(c_hbm.at[pl.ds(tlo, TSLICE)], buf_b)
    pltpu.sync_copy(c_hbm.at[pl.ds(TPAD + tlo, TSLICE)], buf_c)
    pltpu.sync_copy(wh_hbm.at[pl.ds(tlo, TSLICE)], g_v)
    pltpu.sync_copy(se_hbm.at[pl.ds(tlo, TSLICE)], buf_d)

    def g_body(i, _):
        dsl = pl.ds(i * LANES, LANES)
        cnt = jnp.maximum(buf_b[dsl] + buf_c[dsl], 1.0)
        w = g_v[dsl] * buf_d[dsl]
        g_v[dsl] = w * w * buf_a[dsl] / cnt
        return 0

    lax.fori_loop(0, TSLICE // LANES, g_body, 0)
    pltpu.sync_copy(g_v, g_sp.at[pl.ds(tlo, TSLICE)])

    _fill(buf_a, 0.0, TSLICE)
    pltpu.sync_copy(buf_a, o_sp.at[pl.ds(tlo, TSLICE)])
    plsc.subcore_barrier()

    def chunk_body(i, _):
        base = wid * PER_W + i * CHUNK
        pltpu.sync_copy(he_hbm.at[pl.ds(base, CHUNK)], nidx_v)
        pltpu.sync_copy(he_hbm.at[pl.ds(N_CONN + base, CHUNK)], eidx_v)
        pltpu.async_copy(g_sp.at[eidx_v], vals_v, sem).wait()
        pltpu.sync_copy(vals_v, o_sp.at[nidx_v], add=True)
        return 0

    lax.fori_loop(0, N_CHUNKS, chunk_body, 0)
    plsc.subcore_barrier()

    pltpu.sync_copy(o_sp.at[pl.ds(tlo, TSLICE)], buf_a)
    pltpu.sync_copy(buf_a, out_p.at[pl.ds(cid * TPAD + tlo, TSLICE)])


def _combine_body(p_ref, wn_ref, b_ref, o_ref):
    o_ref[...] = ((p_ref[0:1, :] + p_ref[1:2, :])
                  * (W_MAX * W_MAX * wn_ref[0, 0]) + b_ref[0, 0])


def kernel(x_in, hyperedge_index, weight_node, bias, w_hat, s_e):
    x_pad = jnp.zeros((TPAD,), jnp.float32).at[:N_NODES].set(x_in[:, 0])
    wh_pad = jnp.zeros((TPAD,), jnp.float32).at[:N_EDGES].set(w_hat)
    se_pad = jnp.zeros((TPAD,), jnp.float32).at[:N_EDGES].set(s_e)
    he_flat = hyperedge_index.astype(jnp.int32).reshape(2 * N_CONN)

    s_part, c_part = _edge_accum(x_pad, he_flat)
    out_part = _node_scatter(he_flat, s_part, c_part, wh_pad, se_pad)

    out2 = pl.pallas_call(
        _combine_body,
        out_shape=jax.ShapeDtypeStruct((1, TPAD), jnp.float32),
    )(out_part.reshape(NC, TPAD), weight_node, bias.reshape(1, 1))
    return out2[0, :N_NODES, None]
